# Initial kernel scaffold; baseline (speedup 1.0000x reference)
import functools
import math

import jax
import jax.numpy as jnp
from jax import lax
from jax.experimental import pallas as pl
from jax.experimental.pallas import tpu as pltpu
from jax.experimental.pallas import tpu_sc as plsc

N_NODES = 100000
N_EDGES = 1600000
NUM_TYPES = 4
EMB_DIM = 32
BASIS_DIM = 8

_NC = 2
_NS = 16
_NW = _NC * _NS
_EPW = N_EDGES // _NW


def _sc_gather_types(atom_types, edge_index):
    mesh = plsc.VectorSubcoreMesh(core_axis_name="c", subcore_axis_name="s")

    @functools.partial(
        pl.kernel,
        out_type=jax.ShapeDtypeStruct((2, N_EDGES), jnp.int32),
        mesh=mesh,
        scratch_types=[
            pltpu.VMEM((_EPW,), jnp.int32),
            pltpu.VMEM((_EPW,), jnp.int32),
            pltpu.SemaphoreType.DMA,
        ],
    )
    def gather_kernel(atoms_hbm, eidx_hbm, out_hbm, idx_v, t_v, sem):
        wid = lax.axis_index("s") * _NC + lax.axis_index("c")
        base = wid * _EPW
        for r in range(2):
            pltpu.sync_copy(eidx_hbm.at[r, pl.ds(base, _EPW)], idx_v)
            pltpu.async_copy(atoms_hbm.at[idx_v], t_v, sem).wait()
            pltpu.sync_copy(t_v, out_hbm.at[r, pl.ds(base, _EPW)])

    return gather_kernel(atom_types, edge_index)


_BLK = 6400


def _tc_product_kernel(c_ref, n_ref, emb_ref, comb_ref, w_ref, out_ref):
    pair = c_ref[...] * NUM_TYPES + n_ref[...]
    onehot = (pair == lax.broadcasted_iota(jnp.int32, (_BLK, 16), 1))
    type_embed = jnp.dot(onehot.astype(jnp.float32), comb_ref[...],
                         preferred_element_type=jnp.float32)
    alpha = 1.0 / math.sqrt(BASIS_DIM)
    basis = jnp.dot(emb_ref[...], w_ref[...],
                    preferred_element_type=jnp.float32,
                    precision=lax.Precision.HIGHEST) * alpha
    out_ref[...] = type_embed * basis


def _tc_product(types, edge_embedding, comb, basis_W):
    grid = (N_EDGES // _BLK,)
    center = types[0].reshape(N_EDGES, 1)
    neighbor = types[1].reshape(N_EDGES, 1)
    return pl.pallas_call(
        _tc_product_kernel,
        grid=grid,
        in_specs=[
            pl.BlockSpec((_BLK, 1), lambda i: (i, 0)),
            pl.BlockSpec((_BLK, 1), lambda i: (i, 0)),
            pl.BlockSpec((_BLK, BASIS_DIM), lambda i: (i, 0)),
            pl.BlockSpec((16, EMB_DIM), lambda i: (0, 0)),
            pl.BlockSpec((BASIS_DIM, EMB_DIM), lambda i: (0, 0)),
        ],
        out_specs=pl.BlockSpec((_BLK, EMB_DIM), lambda i: (i, 0)),
        out_shape=jax.ShapeDtypeStruct((N_EDGES, EMB_DIM), jnp.float32),
        compiler_params=pltpu.CompilerParams(
            dimension_semantics=("arbitrary",),
        ),
    )(center, neighbor, edge_embedding, comb, basis_W)


def kernel(atom_types, edge_index, edge_embedding, type_embeddings, basis_W):
    types = _sc_gather_types(atom_types.astype(jnp.int32),
                             edge_index.astype(jnp.int32))
    comb = jnp.concatenate(
        [jnp.repeat(type_embeddings[0], NUM_TYPES, axis=0),
         jnp.tile(type_embeddings[1], (NUM_TYPES, 1))], axis=1)
    return _tc_product(types, edge_embedding, comb, basis_W)


# speedup vs baseline: 13.0608x; 13.0608x over previous
import functools
import math

import jax
import jax.numpy as jnp
from jax import lax
from jax.experimental import pallas as pl
from jax.experimental.pallas import tpu as pltpu
from jax.experimental.pallas import tpu_sc as plsc

N_NODES = 100000
N_EDGES = 1600000
NUM_TYPES = 4
EMB_DIM = 32
BASIS_DIM = 8

_NC = 2
_NS = 16
_NW = _NC * _NS
_EPW = N_EDGES // _NW


def _sc_gather_types(atom_types, edge_index_flat):
    mesh = plsc.VectorSubcoreMesh(core_axis_name="c", subcore_axis_name="s")

    @functools.partial(
        pl.kernel,
        out_type=jax.ShapeDtypeStruct((2 * N_EDGES,), jnp.int32),
        mesh=mesh,
        scratch_types=[
            pltpu.VMEM((_EPW,), jnp.int32),
            pltpu.VMEM((_EPW,), jnp.int32),
            pltpu.SemaphoreType.DMA,
        ],
    )
    def gather_kernel(atoms_hbm, eidx_hbm, out_hbm, idx_v, t_v, sem):
        wid = lax.axis_index("s") * _NC + lax.axis_index("c")
        for r in range(2):
            base = r * N_EDGES + wid * _EPW
            pltpu.sync_copy(eidx_hbm.at[pl.ds(base, _EPW)], idx_v)
            pltpu.async_copy(atoms_hbm.at[idx_v], t_v, sem).wait()
            pltpu.sync_copy(t_v, out_hbm.at[pl.ds(base, _EPW)])

    return gather_kernel(atom_types, edge_index_flat)


_BLK = 6400


def _tc_product_kernel(c_ref, n_ref, emb_ref, comb_ref, w_ref, out_ref):
    pair = c_ref[...] * NUM_TYPES + n_ref[...]
    onehot = (pair == lax.broadcasted_iota(jnp.int32, (_BLK, 16), 1))
    type_embed = jnp.dot(onehot.astype(jnp.float32), comb_ref[...],
                         preferred_element_type=jnp.float32)
    alpha = 1.0 / math.sqrt(BASIS_DIM)
    basis = jnp.dot(emb_ref[...], w_ref[...],
                    preferred_element_type=jnp.float32,
                    precision=lax.Precision.HIGHEST) * alpha
    out_ref[...] = type_embed * basis


def _tc_product(types, edge_embedding, comb, basis_W):
    grid = (N_EDGES // _BLK,)
    center = types[0].reshape(N_EDGES, 1)
    neighbor = types[1].reshape(N_EDGES, 1)
    return pl.pallas_call(
        _tc_product_kernel,
        grid=grid,
        in_specs=[
            pl.BlockSpec((_BLK, 1), lambda i: (i, 0)),
            pl.BlockSpec((_BLK, 1), lambda i: (i, 0)),
            pl.BlockSpec((_BLK, BASIS_DIM), lambda i: (i, 0)),
            pl.BlockSpec((16, EMB_DIM), lambda i: (0, 0)),
            pl.BlockSpec((BASIS_DIM, EMB_DIM), lambda i: (0, 0)),
        ],
        out_specs=pl.BlockSpec((_BLK, EMB_DIM), lambda i: (i, 0)),
        out_shape=jax.ShapeDtypeStruct((N_EDGES, EMB_DIM), jnp.float32),
        compiler_params=pltpu.CompilerParams(
            dimension_semantics=("arbitrary",),
        ),
    )(center, neighbor, edge_embedding, comb, basis_W)


def kernel(atom_types, edge_index, edge_embedding, type_embeddings, basis_W):
    types = _sc_gather_types(atom_types.astype(jnp.int32),
                             edge_index.astype(jnp.int32).reshape(-1))
    types = types.reshape(2, N_EDGES)
    comb = jnp.concatenate(
        [jnp.repeat(type_embeddings[0], NUM_TYPES, axis=0),
         jnp.tile(type_embeddings[1], (NUM_TYPES, 1))], axis=1)
    return _tc_product(types, edge_embedding, comb, basis_W)
